# baseline (speedup 1.0000x reference)
import functools

import jax
import jax.numpy as jnp
from jax import lax
from jax.experimental import pallas as pl
from jax.experimental.pallas import tpu as pltpu
from jax.experimental.pallas import tpu_sc as plsc

B_TOK, SEQ = 16384, 50
V, D = 1000000, 64
NC, NS = 2, 16
NW = NC * NS

NCH = 256
N_NCH = B_TOK // NCH
N_UNITS = SEQ * N_NCH
U_PER_W = N_UNITS // NW
NBUF = 2
N_GROUPS = U_PER_W // NBUF

_mesh = plsc.VectorSubcoreMesh(core_axis_name="c", subcore_axis_name="s")


@functools.partial(
    pl.kernel,
    mesh=_mesh,
    out_type=jax.ShapeDtypeStruct((SEQ, D, B_TOK), jnp.float32),
    scratch_types=[
        pltpu.VMEM((NCH,), jnp.int32),
        pltpu.VMEM((NCH,), jnp.int32),
        pltpu.VMEM((NCH, D), jnp.float32),
        pltpu.VMEM((NCH, D), jnp.float32),
        pltpu.VMEM((D, NCH), jnp.float32),
        pltpu.VMEM((D, NCH), jnp.float32),
        pltpu.SemaphoreType.DMA((NBUF,)),
        pltpu.SemaphoreType.DMA((NBUF,)),
    ],
    compiler_params=pltpu.CompilerParams(use_tc_tiling_on_sc=False,
                                         needs_layout_passes=False),
)
def _fused_kernel(table_hbm, idx_hbm, out_hbm,
                  idx0, idx1, rows0, rows1, t0, t1, gsem, wsem):
    wid = lax.axis_index("s") * NC + lax.axis_index("c")
    base_u = wid * U_PER_W
    idx_v = (idx0, idx1)
    rows_v = (rows0, rows1)
    t_v = (t0, t1)

    lane = lax.iota(jnp.int32, 16)
    nvecs = [lane + j * 16 for j in range(NCH // 16)]

    def unit_pos(g):
        u = base_u + g
        return u // N_NCH, (u % N_NCH) * NCH

    def start_gather(g, b):
        s, n0 = unit_pos(g)
        pltpu.sync_copy(idx_hbm.at[s, pl.ds(n0, NCH)], idx_v[b])
        pltpu.async_copy(table_hbm.at[idx_v[b]], rows_v[b], gsem.at[b])

    def wait_gather(b):
        pltpu.make_async_copy(table_hbm.at[idx_v[b]], rows_v[b],
                              gsem.at[b]).wait()

    def transpose(b):
        rows = rows_v[b]
        t = t_v[b]

        def body_c(c, carry):
            cvec = jnp.full((16,), 0, jnp.int32) + c
            for j in range(NCH // 16):
                val = plsc.load_gather(rows, [nvecs[j], cvec])
                t[c, pl.ds(j * 16, 16)] = val
            return carry

        lax.fori_loop(0, D, body_c, 0)

    def start_write(g, b):
        s, n0 = unit_pos(g)
        pltpu.async_copy(t_v[b], out_hbm.at[s, :, pl.ds(n0, NCH)],
                         wsem.at[b])

    def wait_write(g, b):
        s, n0 = unit_pos(g)
        pltpu.make_async_copy(t_v[b], out_hbm.at[s, :, pl.ds(n0, NCH)],
                              wsem.at[b]).wait()

    for b in range(NBUF):
        start_gather(b, b)
    for b in range(NBUF):
        wait_gather(b)
        transpose(b)
        start_write(b, b)
        start_gather(NBUF + b, b)

    def body(i, carry):
        for b in range(NBUF):
            g = i * NBUF + b
            wait_gather(b)
            wait_write(g - NBUF, b)
            transpose(b)
            start_write(g, b)
            start_gather(g + NBUF, b)
        return carry

    lax.fori_loop(1, N_GROUPS - 1, body, 0)

    for b in range(NBUF):
        g = (N_GROUPS - 1) * NBUF + b
        wait_gather(b)
        wait_write(g - NBUF, b)
        transpose(b)
        start_write(g, b)
    for b in range(NBUF):
        g = (N_GROUPS - 1) * NBUF + b
        wait_write(g, b)


V_BLK = 32768
N_VBLK = -(-V // V_BLK)


def _wt_body(wt_ref, o_ref):
    o_ref[...] = wt_ref[...].T


_w_transpose = pl.pallas_call(
    _wt_body,
    grid=(N_VBLK,),
    in_specs=[pl.BlockSpec((D, V_BLK), lambda i: (0, i))],
    out_specs=pl.BlockSpec((V_BLK, D), lambda i: (i, 0)),
    out_shape=jax.ShapeDtypeStruct((V, D), jnp.float32),
)


def kernel(token_ids, weights):
    wt = jnp.swapaxes(weights, 0, 1)
    w_rm = _w_transpose(wt)
    tt = jnp.swapaxes(token_ids, 0, 1)
    out_t = _fused_kernel(w_rm, tt)
    return jnp.transpose(out_t, (2, 0, 1))


# speedup vs baseline: 1.6992x; 1.6992x over previous
import functools

import jax
import jax.numpy as jnp
from jax import lax
from jax.experimental import pallas as pl
from jax.experimental.pallas import tpu as pltpu
from jax.experimental.pallas import tpu_sc as plsc

B_TOK, SEQ = 16384, 50
V, D = 1000000, 64
B = B_TOK * SEQ
NC, NS = 2, 16
NW = NC * NS
B_PER_W = B // NW
CHUNK = 640
SUB = 128
K = CHUNK // SUB
N_CHUNKS = B_PER_W // CHUNK
NBUF = 2
N_GROUPS = N_CHUNKS // NBUF

_mesh = plsc.VectorSubcoreMesh(core_axis_name="c", subcore_axis_name="s")


@functools.partial(
    pl.kernel,
    mesh=_mesh,
    out_type=jax.ShapeDtypeStruct((B, D), jnp.float32),
    scratch_types=[
        pltpu.VMEM((NBUF, CHUNK), jnp.int32),
        pltpu.VMEM((NBUF, CHUNK, D), jnp.float32),
        pltpu.SemaphoreType.DMA((NBUF,)),
        pltpu.SemaphoreType.DMA((NBUF,)),
    ],
    compiler_params=pltpu.CompilerParams(use_tc_tiling_on_sc=False),
)
def _gather_kernel(table_hbm, idx_hbm, out_hbm, idx_v, rows_v, gsem, wsem):
    wid = lax.axis_index("s") * NC + lax.axis_index("c")
    base = wid * B_PER_W

    def fire_gathers(b):
        for j in range(K):
            pltpu.async_copy(
                table_hbm.at[idx_v.at[b, pl.ds(j * SUB, SUB)]],
                rows_v.at[b, pl.ds(j * SUB, SUB)], gsem.at[b])

    def start_gather(ck, b):
        off = base + ck * CHUNK
        pltpu.sync_copy(idx_hbm.at[pl.ds(off, CHUNK)], idx_v.at[b])
        fire_gathers(b)

    for b in range(NBUF):
        start_gather(b, b)

    def body(i, carry):
        for b in range(NBUF):
            ck = i * NBUF + b
            pltpu.make_async_copy(table_hbm.at[idx_v.at[b]], rows_v.at[b],
                                  gsem.at[b]).wait()
            pltpu.async_copy(
                rows_v.at[b], out_hbm.at[pl.ds(base + ck * CHUNK, CHUNK)],
                wsem.at[b])
            nk = ck + NBUF
            off = base + nk * CHUNK
            pltpu.sync_copy(idx_hbm.at[pl.ds(off, CHUNK)], idx_v.at[b])
            pltpu.make_async_copy(
                rows_v.at[b], out_hbm.at[pl.ds(base + ck * CHUNK, CHUNK)],
                wsem.at[b]).wait()
            fire_gathers(b)
        return carry

    lax.fori_loop(0, N_GROUPS - 1, body, 0)

    for b in range(NBUF):
        ck = (N_GROUPS - 1) * NBUF + b
        pltpu.make_async_copy(table_hbm.at[idx_v.at[b]], rows_v.at[b],
                              gsem.at[b]).wait()
        pltpu.async_copy(rows_v.at[b],
                         out_hbm.at[pl.ds(base + ck * CHUNK, CHUNK)],
                         wsem.at[b])
    for b in range(NBUF):
        ck = (N_GROUPS - 1) * NBUF + b
        pltpu.make_async_copy(rows_v.at[b],
                              out_hbm.at[pl.ds(base + ck * CHUNK, CHUNK)],
                              wsem.at[b]).wait()


def kernel(token_ids, weights):
    flat = token_ids.reshape(-1).astype(jnp.int32)
    out = _gather_kernel(weights, flat)
    return out.reshape(B_TOK, SEQ, D)


# speedup vs baseline: 1.7659x; 1.0392x over previous
import functools

import jax
import jax.numpy as jnp
from jax import lax
from jax.experimental import pallas as pl
from jax.experimental.pallas import tpu as pltpu
from jax.experimental.pallas import tpu_sc as plsc

B_TOK, SEQ = 16384, 50
V, D = 1000000, 64
B = B_TOK * SEQ
NC, NS = 2, 16
NW = NC * NS
B_PER_W = B // NW
CHUNK = 640
SUB = 128
K = CHUNK // SUB
N_CHUNKS = B_PER_W // CHUNK
NBUF = 2
N_GROUPS = N_CHUNKS // NBUF

_mesh = plsc.VectorSubcoreMesh(core_axis_name="c", subcore_axis_name="s")


@functools.partial(
    pl.kernel,
    mesh=_mesh,
    out_type=jax.ShapeDtypeStruct((B, D), jnp.float32),
    scratch_types=[
        pltpu.VMEM((NBUF, CHUNK), jnp.int32),
        pltpu.VMEM((NBUF, CHUNK, D), jnp.float32),
        pltpu.SemaphoreType.DMA((NBUF,)),
        pltpu.SemaphoreType.DMA((NBUF,)),
    ],
    compiler_params=pltpu.CompilerParams(use_tc_tiling_on_sc=False),
)
def _gather_kernel(table_hbm, idx_hbm, out_hbm, idx_v, rows_v, gsem, wsem):
    wid = lax.axis_index("s") * NC + lax.axis_index("c")
    base = wid * B_PER_W

    def fire_gathers(b):
        for j in range(K):
            pltpu.async_copy(
                table_hbm.at[idx_v.at[b, pl.ds(j * SUB, SUB)]],
                rows_v.at[b, pl.ds(j * SUB, SUB)], gsem.at[b])

    def start_gather(ck, b):
        off = base + ck * CHUNK
        pltpu.sync_copy(idx_hbm.at[pl.ds(off, CHUNK)], idx_v.at[b])
        fire_gathers(b)

    for b in range(NBUF):
        start_gather(b, b)

    def body(i, carry):
        for b in range(NBUF):
            ck = i * NBUF + b
            pltpu.make_async_copy(table_hbm.at[idx_v.at[b]], rows_v.at[b],
                                  gsem.at[b]).wait()
            pltpu.async_copy(
                rows_v.at[b], out_hbm.at[pl.ds(base + ck * CHUNK, CHUNK)],
                wsem.at[b])
            nk = ck + NBUF
            off = base + nk * CHUNK
            pltpu.sync_copy(idx_hbm.at[pl.ds(off, CHUNK)], idx_v.at[b])
            pltpu.make_async_copy(
                rows_v.at[b], out_hbm.at[pl.ds(base + ck * CHUNK, CHUNK)],
                wsem.at[b]).wait()
            fire_gathers(b)
        return carry

    lax.fori_loop(0, N_GROUPS - 1, body, 0)

    for b in range(NBUF):
        ck = (N_GROUPS - 1) * NBUF + b
        pltpu.make_async_copy(table_hbm.at[idx_v.at[b]], rows_v.at[b],
                              gsem.at[b]).wait()
        pltpu.async_copy(rows_v.at[b],
                         out_hbm.at[pl.ds(base + ck * CHUNK, CHUNK)],
                         wsem.at[b])
    for b in range(NBUF):
        ck = (N_GROUPS - 1) * NBUF + b
        pltpu.make_async_copy(rows_v.at[b],
                              out_hbm.at[pl.ds(base + ck * CHUNK, CHUNK)],
                              wsem.at[b]).wait()


def kernel(token_ids, weights):
    flat = jnp.swapaxes(token_ids, 0, 1).reshape(-1).astype(jnp.int32)
    out = _gather_kernel(weights, flat)
    return jnp.swapaxes(out.reshape(SEQ, B_TOK, D), 0, 1)
